# baseline (speedup 1.0000x reference)
import jax
import jax.numpy as jnp
from jax.experimental import pallas as pl
from jax.experimental.pallas import tpu as pltpu

INPUT_DIM = 512
D_MODEL = 768
PATCH_SIZE = 7.0
MAX_PATCHES = 512
T = 4096
TCHUNK = 512
NCHUNK = T // TCHUNK
BAND = 88


def _segsum_kernel(ids_row_ref, ids_chunk_ref, x_ref,
                   sums_ref, cnt_ref, acc_ref, cacc_ref):
    c = pl.program_id(1)
    hi = ids_row_ref[0, 0, T - 1]
    base_id = hi - (MAX_PATCHES - 1)

    @pl.when(c == 0)
    def _init():
        acc_ref[...] = jnp.zeros_like(acc_ref)
        cacc_ref[...] = jnp.zeros_like(cacc_ref)

    ids_chunk = ids_chunk_ref[0, 0]
    first_rel = ids_chunk_ref[0, 0, 0, 0] - base_id
    start = jnp.clip((first_rel // 8) * 8, 0, MAX_PATCHES - BAND)
    start = pl.multiple_of(start, 8)
    r_iota = jax.lax.broadcasted_iota(jnp.int32, (BAND, TCHUNK), 0)
    sel = (ids_chunk == base_id + start + r_iota)
    s = sel.astype(jnp.bfloat16)

    xc = x_ref[0].astype(jnp.bfloat16)
    acc_ref[pl.ds(start, BAND), :] += jnp.dot(
        s, xc, preferred_element_type=jnp.float32)
    cacc_ref[pl.ds(start, BAND), :] += jnp.sum(
        sel.astype(jnp.float32), axis=1, keepdims=True)

    @pl.when(c == NCHUNK - 1)
    def _flush():
        sums_ref[0] = acc_ref[...]
        cnt_ref[0] = cacc_ref[...]


def _proj_kernel(ids_row_ref, sums_ref, cnt_ref, w_ref, b_ref,
                 out_ref, mask_ref):
    lo = ids_row_ref[0, 0, 0]
    hi = ids_row_ref[0, 0, T - 1]
    num = hi - lo + 1
    feats = sums_ref[0] / jnp.maximum(cnt_ref[0], 1.0)
    proj = jax.lax.dot_general(
        feats.astype(jnp.bfloat16), w_ref[...],
        dimension_numbers=(((1,), (1,)), ((), ())),
        preferred_element_type=jnp.float32,
    )
    out_ref[0] = proj + b_ref[...]
    lane = jax.lax.broadcasted_iota(jnp.int32, (1, MAX_PATCHES), 1)
    mask_ref[0] = ((num - MAX_PATCHES + lane) >= 0).astype(jnp.int32)


@jax.jit
def kernel(x, time_delta, W, b):
    B = x.shape[0]
    t = jnp.cumsum(time_delta, axis=1)
    ids = jnp.floor(t / PATCH_SIZE).astype(jnp.int32)

    ids_row = ids.reshape(B, 1, T)
    ids_chunk = ids.reshape(B, NCHUNK, 1, TCHUNK)

    sums, cnt = pl.pallas_call(
        _segsum_kernel,
        grid=(B, NCHUNK),
        in_specs=[
            pl.BlockSpec((1, 1, T), lambda i, c: (i, 0, 0)),
            pl.BlockSpec((1, 1, 1, TCHUNK), lambda i, c: (i, c, 0, 0)),
            pl.BlockSpec((1, TCHUNK, INPUT_DIM), lambda i, c: (i, c, 0)),
        ],
        out_specs=[
            pl.BlockSpec((1, MAX_PATCHES, INPUT_DIM), lambda i, c: (i, 0, 0)),
            pl.BlockSpec((1, MAX_PATCHES, 1), lambda i, c: (i, 0, 0)),
        ],
        out_shape=[
            jax.ShapeDtypeStruct((B, MAX_PATCHES, INPUT_DIM), jnp.float32),
            jax.ShapeDtypeStruct((B, MAX_PATCHES, 1), jnp.float32),
        ],
        scratch_shapes=[
            pltpu.VMEM((MAX_PATCHES, INPUT_DIM), jnp.float32),
            pltpu.VMEM((MAX_PATCHES, 1), jnp.float32),
        ],
        compiler_params=pltpu.CompilerParams(
            dimension_semantics=("parallel", "arbitrary")),
    )(ids_row, ids_chunk, x)

    out, mask_i32 = pl.pallas_call(
        _proj_kernel,
        grid=(B,),
        in_specs=[
            pl.BlockSpec((1, 1, T), lambda i: (i, 0, 0)),
            pl.BlockSpec((1, MAX_PATCHES, INPUT_DIM), lambda i: (i, 0, 0)),
            pl.BlockSpec((1, MAX_PATCHES, 1), lambda i: (i, 0, 0)),
            pl.BlockSpec((D_MODEL, INPUT_DIM), lambda i: (0, 0)),
            pl.BlockSpec((1, D_MODEL), lambda i: (0, 0)),
        ],
        out_specs=[
            pl.BlockSpec((1, MAX_PATCHES, D_MODEL), lambda i: (i, 0, 0)),
            pl.BlockSpec((1, 1, MAX_PATCHES), lambda i: (i, 0, 0)),
        ],
        out_shape=[
            jax.ShapeDtypeStruct((B, MAX_PATCHES, D_MODEL), jnp.float32),
            jax.ShapeDtypeStruct((B, 1, MAX_PATCHES), jnp.int32),
        ],
        compiler_params=pltpu.CompilerParams(
            dimension_semantics=("parallel",)),
    )(ids_row, sums, cnt, W.astype(jnp.bfloat16), b.reshape(1, D_MODEL))

    masks = mask_i32.reshape(B, MAX_PATCHES) != 0
    return out, masks


# speedup vs baseline: 1.8443x; 1.8443x over previous
import jax
import jax.numpy as jnp
from jax.experimental import pallas as pl
from jax.experimental.pallas import tpu as pltpu

INPUT_DIM = 512
D_MODEL = 768
PATCH_SIZE = 7.0
MAX_PATCHES = 512
T = 4096
TCHUNK = 2048
NCHUNK = T // TCHUNK
SUB = 512
NSUB = TCHUNK // SUB
BAND = 88


def _patch_kernel(ids_row_ref, ids_chunk_ref, x_ref, w_ref, b_ref,
                  out_ref, mask_ref, acc_ref, cnt_ref):
    c = pl.program_id(1)
    lo = ids_row_ref[0, 0, 0]
    hi = ids_row_ref[0, 0, T - 1]
    num = hi - lo + 1
    base_id = hi - (MAX_PATCHES - 1)

    @pl.when(c == 0)
    def _init():
        acc_ref[...] = jnp.zeros_like(acc_ref)
        cnt_ref[...] = jnp.zeros_like(cnt_ref)

    for k in range(NSUB):
        ids_sub = ids_chunk_ref[0, 0, :, pl.ds(k * SUB, SUB)]
        first_rel = ids_chunk_ref[0, 0, 0, k * SUB] - base_id
        start = jnp.clip((first_rel // 8) * 8, 0, MAX_PATCHES - BAND)
        start = pl.multiple_of(start, 8)
        r_iota = jax.lax.broadcasted_iota(jnp.int32, (BAND, SUB), 0)
        sel = (ids_sub == base_id + start + r_iota)
        s = sel.astype(jnp.bfloat16)

        xc = x_ref[0, pl.ds(k * SUB, SUB), :].astype(jnp.bfloat16)
        acc_ref[pl.ds(start, BAND), :] += jnp.dot(
            s, xc, preferred_element_type=jnp.float32)
        cnt_ref[pl.ds(start, BAND), :] += jnp.sum(
            sel.astype(jnp.float32), axis=1, keepdims=True)

    @pl.when(c == NCHUNK - 1)
    def _finish():
        feats = acc_ref[...] / jnp.maximum(cnt_ref[...], 1.0)
        proj = jax.lax.dot_general(
            feats.astype(jnp.bfloat16), w_ref[...],
            dimension_numbers=(((1,), (1,)), ((), ())),
            preferred_element_type=jnp.float32,
        )
        out_ref[0] = proj + b_ref[...]
        lane = jax.lax.broadcasted_iota(jnp.int32, (1, MAX_PATCHES), 1)
        mask_ref[0] = ((num - MAX_PATCHES + lane) >= 0).astype(jnp.int32)


@jax.jit
def kernel(x, time_delta, W, b):
    B = x.shape[0]
    t = jnp.cumsum(time_delta, axis=1)
    ids = jnp.floor(t / PATCH_SIZE).astype(jnp.int32)

    ids_row = ids.reshape(B, 1, T)
    ids_chunk = ids.reshape(B, NCHUNK, 1, TCHUNK)

    out, mask_i32 = pl.pallas_call(
        _patch_kernel,
        grid=(B, NCHUNK),
        in_specs=[
            pl.BlockSpec((1, 1, T), lambda i, c: (i, 0, 0)),
            pl.BlockSpec((1, 1, 1, TCHUNK), lambda i, c: (i, c, 0, 0)),
            pl.BlockSpec((1, TCHUNK, INPUT_DIM), lambda i, c: (i, c, 0)),
            pl.BlockSpec((D_MODEL, INPUT_DIM), lambda i, c: (0, 0)),
            pl.BlockSpec((1, D_MODEL), lambda i, c: (0, 0)),
        ],
        out_specs=[
            pl.BlockSpec((1, MAX_PATCHES, D_MODEL), lambda i, c: (i, 0, 0)),
            pl.BlockSpec((1, 1, MAX_PATCHES), lambda i, c: (i, 0, 0)),
        ],
        out_shape=[
            jax.ShapeDtypeStruct((B, MAX_PATCHES, D_MODEL), jnp.float32),
            jax.ShapeDtypeStruct((B, 1, MAX_PATCHES), jnp.int32),
        ],
        scratch_shapes=[
            pltpu.VMEM((MAX_PATCHES, INPUT_DIM), jnp.float32),
            pltpu.VMEM((MAX_PATCHES, 1), jnp.float32),
        ],
        compiler_params=pltpu.CompilerParams(
            dimension_semantics=("parallel", "arbitrary")),
    )(ids_row, ids_chunk, x, W.astype(jnp.bfloat16), b.reshape(1, D_MODEL))

    masks = mask_i32.reshape(B, MAX_PATCHES) != 0
    return out, masks


# speedup vs baseline: 2.1757x; 1.1797x over previous
import jax
import jax.numpy as jnp
from jax.experimental import pallas as pl
from jax.experimental.pallas import tpu as pltpu

INPUT_DIM = 512
D_MODEL = 768
PATCH_SIZE = 7.0
MAX_PATCHES = 512
T = 4096
TCHUNK = 4096
NCHUNK = T // TCHUNK
SUB = 512
NSUB = TCHUNK // SUB
BAND = 88


def _patch_kernel(ids_row_ref, ids_chunk_ref, x_ref, w_ref, b_ref,
                  out_ref, mask_ref, acc_ref, cnt_ref):
    c = pl.program_id(1)
    lo = ids_row_ref[0, 0, 0]
    hi = ids_row_ref[0, 0, T - 1]
    num = hi - lo + 1
    base_id = hi - (MAX_PATCHES - 1)

    @pl.when(c == 0)
    def _init():
        acc_ref[...] = jnp.zeros_like(acc_ref)
        cnt_ref[...] = jnp.zeros_like(cnt_ref)

    for k in range(NSUB):
        ids_sub = ids_chunk_ref[0, 0, :, pl.ds(k * SUB, SUB)]
        first_rel = ids_chunk_ref[0, 0, 0, k * SUB] - base_id
        start = jnp.clip((first_rel // 8) * 8, 0, MAX_PATCHES - BAND)
        start = pl.multiple_of(start, 8)
        r_iota = jax.lax.broadcasted_iota(jnp.int32, (BAND, SUB), 0)
        sel = (ids_sub == base_id + start + r_iota)
        s = sel.astype(jnp.bfloat16)

        xc = x_ref[0, pl.ds(k * SUB, SUB), :].astype(jnp.bfloat16)
        acc_ref[pl.ds(start, BAND), :] += jnp.dot(
            s, xc, preferred_element_type=jnp.float32)
        cnt_ref[pl.ds(start, BAND), :] += jnp.sum(
            sel.astype(jnp.float32), axis=1, keepdims=True)

    @pl.when(c == NCHUNK - 1)
    def _finish():
        feats = acc_ref[...] / jnp.maximum(cnt_ref[...], 1.0)
        proj = jax.lax.dot_general(
            feats.astype(jnp.bfloat16), w_ref[...],
            dimension_numbers=(((1,), (1,)), ((), ())),
            preferred_element_type=jnp.float32,
        )
        out_ref[0] = proj + b_ref[...]
        lane = jax.lax.broadcasted_iota(jnp.int32, (1, MAX_PATCHES), 1)
        mask_ref[0] = ((num - MAX_PATCHES + lane) >= 0).astype(jnp.int32)


@jax.jit
def kernel(x, time_delta, W, b):
    B = x.shape[0]
    t = jnp.cumsum(time_delta, axis=1)
    ids = jnp.floor(t / PATCH_SIZE).astype(jnp.int32)

    ids_row = ids.reshape(B, 1, T)
    ids_chunk = ids.reshape(B, NCHUNK, 1, TCHUNK)

    out, mask_i32 = pl.pallas_call(
        _patch_kernel,
        grid=(B, NCHUNK),
        in_specs=[
            pl.BlockSpec((1, 1, T), lambda i, c: (i, 0, 0)),
            pl.BlockSpec((1, 1, 1, TCHUNK), lambda i, c: (i, c, 0, 0)),
            pl.BlockSpec((1, TCHUNK, INPUT_DIM), lambda i, c: (i, c, 0)),
            pl.BlockSpec((D_MODEL, INPUT_DIM), lambda i, c: (0, 0)),
            pl.BlockSpec((1, D_MODEL), lambda i, c: (0, 0)),
        ],
        out_specs=[
            pl.BlockSpec((1, MAX_PATCHES, D_MODEL), lambda i, c: (i, 0, 0)),
            pl.BlockSpec((1, 1, MAX_PATCHES), lambda i, c: (i, 0, 0)),
        ],
        out_shape=[
            jax.ShapeDtypeStruct((B, MAX_PATCHES, D_MODEL), jnp.float32),
            jax.ShapeDtypeStruct((B, 1, MAX_PATCHES), jnp.int32),
        ],
        scratch_shapes=[
            pltpu.VMEM((MAX_PATCHES, INPUT_DIM), jnp.float32),
            pltpu.VMEM((MAX_PATCHES, 1), jnp.float32),
        ],
        compiler_params=pltpu.CompilerParams(
            dimension_semantics=("parallel", "arbitrary")),
    )(ids_row, ids_chunk, x, W.astype(jnp.bfloat16), b.reshape(1, D_MODEL))

    masks = mask_i32.reshape(B, MAX_PATCHES) != 0
    return out, masks
